# Initial kernel scaffold; baseline (speedup 1.0000x reference)
import jax
import jax.numpy as jnp
from jax.experimental import pallas as pl


def kernel(world_pos, prev_world_pos, target_world_pos, mesh_pos, node_type, cells, params):
    raise NotImplementedError("write your pallas kernel here")


# speedup vs baseline: 1.9657x; 1.9657x over previous
import functools

import jax
import jax.numpy as jnp
from jax import lax
from jax.experimental import pallas as pl
from jax.experimental.pallas import tpu as pltpu
from jax.experimental.pallas import tpu_sc as plsc

N = 10000
C = 20000
E = 6 * C
L = 128
STEPS = 15
NC, NS = 2, 16
NW = NC * NS
CHUNK = 128
NCH = 30
E_PAD = NW * NCH * CHUNK
AGG_ROWS = 10240
ROWS_PER_TILE = AGG_ROWS // NS
JUNK = AGG_ROWS - 1

_f32 = jnp.float32


def _sc_mesh():
    return plsc.VectorSubcoreMesh(core_axis_name="c", subcore_axis_name="s")


@functools.lru_cache(maxsize=None)
def _make_sc_gather_pair(width):
    @functools.partial(
        pl.kernel,
        out_type=(jax.ShapeDtypeStruct((E_PAD, width), _f32),
                  jax.ShapeDtypeStruct((E_PAD, width), _f32)),
        mesh=_sc_mesh(),
        scratch_types=[
            pltpu.VMEM((NCH, CHUNK), jnp.int32),
            pltpu.VMEM((NCH, CHUNK), jnp.int32),
            pltpu.VMEM((CHUNK, width), _f32),
            pltpu.VMEM((CHUNK, width), _f32),
            pltpu.SemaphoreType.DMA,
            pltpu.SemaphoreType.DMA,
        ],
    )
    def gather_pair(ta_hbm, tb_hbm, si_hbm, di_hbm, oa_hbm, ob_hbm,
                    si_v, di_v, bufa, bufb, sema, semb):
        wid = lax.axis_index("s") * NC + lax.axis_index("c")
        base = wid * (NCH * CHUNK)
        pltpu.sync_copy(si_hbm.at[wid], si_v)
        pltpu.sync_copy(di_hbm.at[wid], di_v)

        def body(j, carry):
            ca = pltpu.async_copy(ta_hbm.at[si_v.at[j]], bufa, sema)
            cb = pltpu.async_copy(tb_hbm.at[di_v.at[j]], bufb, semb)
            ca.wait()
            cb.wait()
            pltpu.sync_copy(bufa, oa_hbm.at[pl.ds(base + j * CHUNK, CHUNK)])
            pltpu.sync_copy(bufb, ob_hbm.at[pl.ds(base + j * CHUNK, CHUNK)])
            return carry

        lax.fori_loop(0, NCH, body, 0)

    return gather_pair


@functools.lru_cache(maxsize=None)
def _make_sc_scatter_add():
    @functools.partial(
        pl.kernel,
        out_type=jax.ShapeDtypeStruct((NC, AGG_ROWS, L), _f32),
        mesh=_sc_mesh(),
        scratch_types=[
            pltpu.VMEM((NCH, CHUNK), jnp.int32),
            pltpu.VMEM((CHUNK, L), _f32),
            pltpu.VMEM_SHARED((AGG_ROWS, L), _f32),
        ],
    )
    def scatter_add(el_hbm, di_hbm, zero_hbm, agg_hbm, di_v, buf, acc_sh):
        c = lax.axis_index("c")
        s = lax.axis_index("s")
        wid = s * NC + c
        base = wid * (NCH * CHUNK)
        row0 = s * ROWS_PER_TILE
        pltpu.sync_copy(zero_hbm, acc_sh.at[pl.ds(row0, ROWS_PER_TILE)])
        pltpu.sync_copy(di_hbm.at[wid], di_v)
        plsc.subcore_barrier()

        def body(j, carry):
            pltpu.sync_copy(el_hbm.at[pl.ds(base + j * CHUNK, CHUNK)], buf)
            pltpu.sync_copy(buf, acc_sh.at[di_v.at[j]], add=True)
            return carry

        lax.fori_loop(0, NCH, body, 0)
        plsc.subcore_barrier()
        pltpu.sync_copy(acc_sh.at[pl.ds(row0, ROWS_PER_TILE)],
                        agg_hbm.at[c, pl.ds(row0, ROWS_PER_TILE)])

    return scatter_add


def _dot(a, b):
    return jnp.dot(a, b, preferred_element_type=_f32)


def _ln(y, g, b):
    m = jnp.mean(y, axis=-1, keepdims=True)
    v = jnp.mean((y - m) ** 2, axis=-1, keepdims=True)
    return (y - m) * lax.rsqrt(v + 1e-5) * g + b


def _full(shape):
    return pl.BlockSpec(shape, lambda i: (0,) * len(shape))


def _rows(block_rows, cols):
    return pl.BlockSpec((block_rows, cols), lambda i: (i, 0))


BE = 1024
BN = 1000


def _edge_update_body(x_ref, gs_ref, gd_ref, w1, w2, w3, b2, b3, g, be, o_ref):
    x = x_ref[...]
    h = jnp.maximum(_dot(x, w1[...]) + gs_ref[...] + gd_ref[...], 0.0)
    h = jnp.maximum(_dot(h, w2[...]) + b2[...], 0.0)
    y = _dot(h, w3[...]) + b3[...]
    o_ref[...] = x + _ln(y, g[...], be[...])


def _edge_update(el, gs, gd, p):
    w1e = p['W1'][0:L]
    return pl.pallas_call(
        _edge_update_body,
        grid=(E_PAD // BE,),
        in_specs=[_rows(BE, L), _rows(BE, L), _rows(BE, L),
                  _full((L, L)), _full((L, L)), _full((L, L)),
                  _full((1, L)), _full((1, L)), _full((1, L)), _full((1, L))],
        out_specs=_rows(BE, L),
        out_shape=jax.ShapeDtypeStruct((E_PAD, L), _f32),
    )(el, gs, gd, w1e, p['W2'], p['W3'], p['b2'][None], p['b3'][None],
      p['g'][None], p['be'][None])


def _node_update_body(x_ref, a0_ref, a1_ref, w1n, w1a, b1, w2, w3, b2, b3, g,
                      be, ws, wd, bd, o_ref, ps_ref, pd_ref, *, proj):
    x = x_ref[...]
    a = a0_ref[...] + a1_ref[...]
    h = jnp.maximum(_dot(x, w1n[...]) + _dot(a, w1a[...]) + b1[...], 0.0)
    h = jnp.maximum(_dot(h, w2[...]) + b2[...], 0.0)
    y = _dot(h, w3[...]) + b3[...]
    o = x + _ln(y, g[...], be[...])
    o_ref[...] = o
    if proj:
        ps_ref[...] = _dot(o, ws[...])
        pd_ref[...] = _dot(o, wd[...]) + bd[...]


def _node_update(nl, agg0, agg1, p, pe_next):
    proj = pe_next is not None
    w1n = p['W1'][0:L]
    w1a = p['W1'][L:2 * L]
    if proj:
        ws = pe_next['W1'][L:2 * L]
        wd = pe_next['W1'][2 * L:3 * L]
        bd = pe_next['b1'][None]
        n_out = 3
    else:
        ws = jnp.zeros((1, L), _f32)
        wd = jnp.zeros((1, L), _f32)
        bd = jnp.zeros((1, L), _f32)
        n_out = 1
    out_shapes = [jax.ShapeDtypeStruct((N, L), _f32)] * n_out
    out_specs = [_rows(BN, L)] * n_out

    def body(*refs):
        if proj:
            _node_update_body(*refs, proj=True)
        else:
            x_refs = refs[:15]
            _node_update_body(*x_refs, refs[15], None, None, proj=False)

    res = pl.pallas_call(
        body,
        grid=(N // BN,),
        in_specs=[_rows(BN, L), _rows(BN, L), _rows(BN, L),
                  _full((L, L)), _full((L, L)), _full((1, L)),
                  _full((L, L)), _full((L, L)), _full((1, L)), _full((1, L)),
                  _full((1, L)), _full((1, L)),
                  _full(ws.shape), _full(wd.shape), _full((1, L))],
        out_specs=out_specs if proj else out_specs[0],
        out_shape=out_shapes if proj else out_shapes[0],
    )(nl, agg0, agg1, w1n, w1a, p['b1'][None], p['W2'], p['W3'],
      p['b2'][None], p['b3'][None], p['g'][None], p['be'][None], ws, wd, bd)
    if proj:
        return res
    return res, None, None


def _node_encoder_body(wp_ref, pwp_ref, nt_ref, mean, std, w1, b1, w2, w3, b2,
                       b3, g, be, ws, wd, bd, o_ref, ps_ref, pd_ref):
    vel = wp_ref[...] - pwp_ref[...]
    t = nt_ref[...]
    oh = (t == lax.broadcasted_iota(jnp.int32, (1, 9), 1)).astype(_f32)
    f = jnp.concatenate([vel, oh], axis=-1)
    f = (f - mean[...]) / std[...]
    h = jnp.maximum(_dot(f, w1[...]) + b1[...], 0.0)
    h = jnp.maximum(_dot(h, w2[...]) + b2[...], 0.0)
    y = _dot(h, w3[...]) + b3[...]
    o = _ln(y, g[...], be[...])
    o_ref[...] = o
    ps_ref[...] = _dot(o, ws[...])
    pd_ref[...] = _dot(o, wd[...]) + bd[...]


def _node_encoder(world_pos, prev_world_pos, node_type, params):
    p = params['node_enc']
    pe0 = params['proc_edge'][0]
    return pl.pallas_call(
        _node_encoder_body,
        grid=(N // BN,),
        in_specs=[_rows(BN, 3), _rows(BN, 3), _rows(BN, 1),
                  _full((1, 12)), _full((1, 12)),
                  _full((12, L)), _full((1, L)),
                  _full((L, L)), _full((L, L)), _full((1, L)), _full((1, L)),
                  _full((1, L)), _full((1, L)),
                  _full((L, L)), _full((L, L)), _full((1, L))],
        out_specs=[_rows(BN, L)] * 3,
        out_shape=[jax.ShapeDtypeStruct((N, L), _f32)] * 3,
    )(world_pos, prev_world_pos, node_type[:, None],
      params['node_mean'][None], params['node_std'][None],
      p['W1'], p['b1'][None], p['W2'], p['W3'], p['b2'][None], p['b3'][None],
      p['g'][None], p['be'][None],
      pe0['W1'][L:2 * L], pe0['W1'][2 * L:3 * L], pe0['b1'][None])


def _edge_encoder_body(ts_ref, td_ref, mean, std, w1, b1, w2, w3, b2, b3, g,
                       be, o_ref):
    d = ts_ref[:, 0:5] - td_ref[:, 0:5]
    dm = d[:, 0:2]
    dw = d[:, 2:5]
    nm = jnp.sqrt(jnp.sum(dm * dm, axis=-1, keepdims=True))
    nw = jnp.sqrt(jnp.sum(dw * dw, axis=-1, keepdims=True))
    f = jnp.concatenate([dm, nm, dw, nw], axis=-1)
    f = (f - mean[...]) / std[...]
    h = jnp.maximum(_dot(f, w1[...]) + b1[...], 0.0)
    h = jnp.maximum(_dot(h, w2[...]) + b2[...], 0.0)
    y = _dot(h, w3[...]) + b3[...]
    o_ref[...] = _ln(y, g[...], be[...])


def _edge_encoder(ts, td, params):
    p = params['edge_enc']
    return pl.pallas_call(
        _edge_encoder_body,
        grid=(E_PAD // BE,),
        in_specs=[_rows(BE, L), _rows(BE, L),
                  _full((1, 7)), _full((1, 7)),
                  _full((7, L)), _full((1, L)),
                  _full((L, L)), _full((L, L)), _full((1, L)), _full((1, L)),
                  _full((1, L)), _full((1, L))],
        out_specs=_rows(BE, L),
        out_shape=jax.ShapeDtypeStruct((E_PAD, L), _f32),
    )(ts, td, params['edge_mean'][None], params['edge_std'][None],
      p['W1'], p['b1'][None], p['W2'], p['W3'], p['b2'][None], p['b3'][None],
      p['g'][None], p['be'][None])


def _decoder_body(x_ref, w1, b1, w2, b2, w3, b3, ostd, omean, o_ref):
    h = jnp.maximum(_dot(x_ref[...], w1[...]) + b1[...], 0.0)
    h = jnp.maximum(_dot(h, w2[...]) + b2[...], 0.0)
    y = _dot(h, w3[...]) + b3[...]
    o_ref[...] = y * ostd[...] + omean[...]


def _decoder(nl, params):
    p = params['decoder']
    w3p = jnp.zeros((L, L), _f32).at[:, 0:3].set(p['W3'])
    b3p = jnp.zeros((1, L), _f32).at[:, 0:3].set(p['b3'][None])
    ostd = jnp.ones((1, L), _f32).at[:, 0:3].set(params['out_std'][None])
    omean = jnp.zeros((1, L), _f32).at[:, 0:3].set(params['out_mean'][None])
    out = pl.pallas_call(
        _decoder_body,
        grid=(N // BN,),
        in_specs=[_rows(BN, L),
                  _full((L, L)), _full((1, L)), _full((L, L)), _full((1, L)),
                  _full((L, L)), _full((1, L)), _full((1, L)), _full((1, L))],
        out_specs=_rows(BN, L),
        out_shape=jax.ShapeDtypeStruct((N, L), _f32),
    )(nl, p['W1'], p['b1'][None], p['W2'], p['b2'][None], w3p, b3p, ostd,
      omean)
    return out[:, 0:3]


def kernel(world_pos, prev_world_pos, target_world_pos, mesh_pos, node_type,
           cells, params):
    del target_world_pos
    a, b, c = cells[:, 0], cells[:, 1], cells[:, 2]
    srcs = jnp.concatenate([a, b, c, b, c, a]).astype(jnp.int32)
    dsts = jnp.concatenate([b, c, a, a, b, c]).astype(jnp.int32)
    pad = E_PAD - E
    si = jnp.pad(srcs, (0, pad)).reshape(NW, NCH, CHUNK)
    di_g = jnp.pad(dsts, (0, pad)).reshape(NW, NCH, CHUNK)
    di_s = jnp.pad(dsts, (0, pad), constant_values=JUNK).reshape(NW, NCH, CHUNK)

    tbl = jnp.concatenate(
        [mesh_pos, world_pos, jnp.zeros((N, L - 5), _f32)], axis=1)
    ts, td = _make_sc_gather_pair(L)(tbl, tbl, si, di_g)
    el = _edge_encoder(ts, td, params)
    nl, ps, pd = _node_encoder(world_pos, prev_world_pos, node_type, params)

    zero = jnp.zeros((ROWS_PER_TILE, L), _f32)
    gather = _make_sc_gather_pair(L)
    scatter = _make_sc_scatter_add()
    for i in range(STEPS):
        gs, gd = gather(ps, pd, si, di_g)
        el = _edge_update(el, gs, gd, params['proc_edge'][i])
        agg = scatter(el, di_s, zero)
        pe_next = params['proc_edge'][i + 1] if i + 1 < STEPS else None
        nl, ps, pd = _node_update(nl, agg[0, :N], agg[1, :N],
                                  params['proc_node'][i], pe_next)
    return _decoder(nl, params)


# speedup vs baseline: 2.1508x; 1.0941x over previous
import functools

import jax
import jax.numpy as jnp
from jax import lax
from jax.experimental import pallas as pl
from jax.experimental.pallas import tpu as pltpu
from jax.experimental.pallas import tpu_sc as plsc

N = 10000
C = 20000
E = 6 * C
L = 128
STEPS = 15
NC, NS = 2, 16
NW = NC * NS
CHUNK = 96
NCH = 40
NSLOT = 4
E_PAD = NW * NCH * CHUNK
AGG_ROWS = 10240
ROWS_PER_TILE = AGG_ROWS // NS
JUNK = AGG_ROWS - 1

_f32 = jnp.float32


def _sc_mesh():
    return plsc.VectorSubcoreMesh(core_axis_name="c", subcore_axis_name="s")


@functools.lru_cache(maxsize=None)
def _make_sc_gather_pair(width):
    @functools.partial(
        pl.kernel,
        out_type=(jax.ShapeDtypeStruct((E_PAD, width), _f32),
                  jax.ShapeDtypeStruct((E_PAD, width), _f32)),
        mesh=_sc_mesh(),
        scratch_types=(
            [pltpu.VMEM((NCH, CHUNK), jnp.int32)] * 2
            + [pltpu.VMEM((CHUNK, width), _f32)] * (2 * NSLOT)
            + [pltpu.SemaphoreType.DMA] * (2 * NSLOT)
        ),
    )
    def gather_pair(ta_hbm, tb_hbm, si_hbm, di_hbm, oa_hbm, ob_hbm,
                    si_v, di_v, *bufs_sems):
        bufa = bufs_sems[0:NSLOT]
        bufb = bufs_sems[NSLOT:2 * NSLOT]
        sg = bufs_sems[2 * NSLOT:3 * NSLOT]
        sw = bufs_sems[3 * NSLOT:4 * NSLOT]
        wid = lax.axis_index("s") * NC + lax.axis_index("c")
        base = wid * (NCH * CHUNK)
        pltpu.sync_copy(si_hbm.at[wid], si_v)
        pltpu.sync_copy(di_hbm.at[wid], di_v)

        def g_start(slot, v):
            pltpu.async_copy(ta_hbm.at[si_v.at[v]], bufa[slot], sg[slot])
            pltpu.async_copy(tb_hbm.at[di_v.at[v]], bufb[slot], sg[slot])

        def g_wait(slot):
            pltpu.make_async_copy(ta_hbm.at[si_v.at[0]], bufa[slot],
                                  sg[slot]).wait()
            pltpu.make_async_copy(tb_hbm.at[di_v.at[0]], bufb[slot],
                                  sg[slot]).wait()

        def w_start(slot, v):
            dst = pl.ds(base + v * CHUNK, CHUNK)
            pltpu.async_copy(bufa[slot], oa_hbm.at[dst], sw[slot])
            pltpu.async_copy(bufb[slot], ob_hbm.at[dst], sw[slot])

        def w_wait(slot):
            dst = pl.ds(base, CHUNK)
            pltpu.make_async_copy(bufa[slot], oa_hbm.at[dst], sw[slot]).wait()
            pltpu.make_async_copy(bufb[slot], ob_hbm.at[dst], sw[slot]).wait()

        g_start(0, 0)
        g_start(1, 1)
        g_wait(0)
        w_start(0, 0)
        g_start(2, 2)
        g_wait(1)
        w_start(1, 1)
        g_start(3, 3)

        def body(i, carry):
            for b in range(4):
                v = 2 + i * 4 + b
                slot = (2 + b) % 4
                slot2 = (slot + 2) % 4
                g_wait(slot)
                w_start(slot, v)
                w_wait(slot2)
                g_start(slot2, v + 2)
            return carry

        lax.fori_loop(0, (NCH - 4) // 4, body, 0)
        g_wait((NCH - 2) % 4)
        w_start((NCH - 2) % 4, NCH - 2)
        g_wait((NCH - 1) % 4)
        w_start((NCH - 1) % 4, NCH - 1)
        for slot in range(4):
            w_wait(slot)

    return gather_pair


@functools.lru_cache(maxsize=None)
def _make_sc_scatter_add():
    @functools.partial(
        pl.kernel,
        out_type=jax.ShapeDtypeStruct((NC, AGG_ROWS, L), _f32),
        mesh=_sc_mesh(),
        scratch_types=(
            [pltpu.VMEM((NCH, CHUNK), jnp.int32)]
            + [pltpu.VMEM((CHUNK, L), _f32)] * 2
            + [pltpu.SemaphoreType.DMA] * 4
            + [pltpu.VMEM_SHARED((AGG_ROWS, L), _f32)]
        ),
    )
    def scatter_add(el_hbm, di_hbm, zero_hbm, agg_hbm, di_v, *bufs_sems):
        buf = bufs_sems[0:2]
        sl = bufs_sems[2:4]
        sa = bufs_sems[4:6]
        acc_sh = bufs_sems[6]
        c = lax.axis_index("c")
        s = lax.axis_index("s")
        wid = s * NC + c
        base = wid * (NCH * CHUNK)
        row0 = s * ROWS_PER_TILE
        pltpu.sync_copy(di_hbm.at[wid], di_v)
        pltpu.sync_copy(zero_hbm, acc_sh.at[pl.ds(row0, ROWS_PER_TILE)])
        plsc.subcore_barrier()

        def l_start(slot, v):
            pltpu.async_copy(el_hbm.at[pl.ds(base + v * CHUNK, CHUNK)],
                             buf[slot], sl[slot])

        def l_wait(slot):
            pltpu.make_async_copy(el_hbm.at[pl.ds(base, CHUNK)], buf[slot],
                                  sl[slot]).wait()

        def a_start(slot, v):
            pltpu.async_copy(buf[slot], acc_sh.at[di_v.at[v]], sa[slot],
                             add=True)

        def a_wait(slot):
            pltpu.make_async_copy(buf[slot], acc_sh.at[di_v.at[0]],
                                  sa[slot]).wait()

        l_start(0, 0)
        l_start(1, 1)
        l_wait(0)
        a_start(0, 0)

        def body(i, carry):
            for b_off in range(2):
                v = 1 + i * 2 + b_off
                slot = (1 + b_off) % 2
                other = 1 - slot
                l_wait(slot)
                a_start(slot, v)
                a_wait(other)
                l_start(other, v + 1)
            return carry

        lax.fori_loop(0, (NCH - 2) // 2, body, 0)
        l_wait((NCH - 1) % 2)
        a_start((NCH - 1) % 2, NCH - 1)
        a_wait((NCH - 2) % 2)
        a_wait((NCH - 1) % 2)
        plsc.subcore_barrier()
        pltpu.sync_copy(acc_sh.at[pl.ds(row0, ROWS_PER_TILE)],
                        agg_hbm.at[c, pl.ds(row0, ROWS_PER_TILE)])

    return scatter_add


def _dot(a, b):
    return jnp.dot(a, b, preferred_element_type=_f32)


def _ln(y, g, b):
    m = jnp.mean(y, axis=-1, keepdims=True)
    v = jnp.mean((y - m) ** 2, axis=-1, keepdims=True)
    return (y - m) * lax.rsqrt(v + 1e-5) * g + b


def _full(shape):
    return pl.BlockSpec(shape, lambda i: (0,) * len(shape))


def _rows(block_rows, cols):
    return pl.BlockSpec((block_rows, cols), lambda i: (i, 0))


BE = 1024
BN = 1000


def _edge_update_body(x_ref, gs_ref, gd_ref, w1, w2, w3, b2, b3, g, be, o_ref):
    x = x_ref[...]
    h = jnp.maximum(_dot(x, w1[...]) + gs_ref[...] + gd_ref[...], 0.0)
    h = jnp.maximum(_dot(h, w2[...]) + b2[...], 0.0)
    y = _dot(h, w3[...]) + b3[...]
    o_ref[...] = x + _ln(y, g[...], be[...])


def _edge_update(el, gs, gd, p):
    w1e = p['W1'][0:L]
    return pl.pallas_call(
        _edge_update_body,
        grid=(E_PAD // BE,),
        in_specs=[_rows(BE, L), _rows(BE, L), _rows(BE, L),
                  _full((L, L)), _full((L, L)), _full((L, L)),
                  _full((1, L)), _full((1, L)), _full((1, L)), _full((1, L))],
        out_specs=_rows(BE, L),
        out_shape=jax.ShapeDtypeStruct((E_PAD, L), _f32),
    )(el, gs, gd, w1e, p['W2'], p['W3'], p['b2'][None], p['b3'][None],
      p['g'][None], p['be'][None])


def _node_update_body(x_ref, a0_ref, a1_ref, w1n, w1a, b1, w2, w3, b2, b3, g,
                      be, ws, wd, bd, o_ref, ps_ref, pd_ref, *, proj):
    x = x_ref[...]
    a = a0_ref[...] + a1_ref[...]
    h = jnp.maximum(_dot(x, w1n[...]) + _dot(a, w1a[...]) + b1[...], 0.0)
    h = jnp.maximum(_dot(h, w2[...]) + b2[...], 0.0)
    y = _dot(h, w3[...]) + b3[...]
    o = x + _ln(y, g[...], be[...])
    o_ref[...] = o
    if proj:
        ps_ref[...] = _dot(o, ws[...])
        pd_ref[...] = _dot(o, wd[...]) + bd[...]


def _node_update(nl, agg0, agg1, p, pe_next):
    proj = pe_next is not None
    w1n = p['W1'][0:L]
    w1a = p['W1'][L:2 * L]
    if proj:
        ws = pe_next['W1'][L:2 * L]
        wd = pe_next['W1'][2 * L:3 * L]
        bd = pe_next['b1'][None]
        n_out = 3
    else:
        ws = jnp.zeros((1, L), _f32)
        wd = jnp.zeros((1, L), _f32)
        bd = jnp.zeros((1, L), _f32)
        n_out = 1
    out_shapes = [jax.ShapeDtypeStruct((N, L), _f32)] * n_out
    out_specs = [_rows(BN, L)] * n_out

    def body(*refs):
        if proj:
            _node_update_body(*refs, proj=True)
        else:
            x_refs = refs[:15]
            _node_update_body(*x_refs, refs[15], None, None, proj=False)

    res = pl.pallas_call(
        body,
        grid=(N // BN,),
        in_specs=[_rows(BN, L), _rows(BN, L), _rows(BN, L),
                  _full((L, L)), _full((L, L)), _full((1, L)),
                  _full((L, L)), _full((L, L)), _full((1, L)), _full((1, L)),
                  _full((1, L)), _full((1, L)),
                  _full(ws.shape), _full(wd.shape), _full((1, L))],
        out_specs=out_specs if proj else out_specs[0],
        out_shape=out_shapes if proj else out_shapes[0],
    )(nl, agg0, agg1, w1n, w1a, p['b1'][None], p['W2'], p['W3'],
      p['b2'][None], p['b3'][None], p['g'][None], p['be'][None], ws, wd, bd)
    if proj:
        return res
    return res, None, None


def _node_encoder_body(wp_ref, pwp_ref, nt_ref, mean, std, w1, b1, w2, w3, b2,
                       b3, g, be, ws, wd, bd, o_ref, ps_ref, pd_ref):
    vel = wp_ref[...] - pwp_ref[...]
    t = nt_ref[...]
    oh = (t == lax.broadcasted_iota(jnp.int32, (1, 9), 1)).astype(_f32)
    f = jnp.concatenate([vel, oh], axis=-1)
    f = (f - mean[...]) / std[...]
    h = jnp.maximum(_dot(f, w1[...]) + b1[...], 0.0)
    h = jnp.maximum(_dot(h, w2[...]) + b2[...], 0.0)
    y = _dot(h, w3[...]) + b3[...]
    o = _ln(y, g[...], be[...])
    o_ref[...] = o
    ps_ref[...] = _dot(o, ws[...])
    pd_ref[...] = _dot(o, wd[...]) + bd[...]


def _node_encoder(world_pos, prev_world_pos, node_type, params):
    p = params['node_enc']
    pe0 = params['proc_edge'][0]
    return pl.pallas_call(
        _node_encoder_body,
        grid=(N // BN,),
        in_specs=[_rows(BN, 3), _rows(BN, 3), _rows(BN, 1),
                  _full((1, 12)), _full((1, 12)),
                  _full((12, L)), _full((1, L)),
                  _full((L, L)), _full((L, L)), _full((1, L)), _full((1, L)),
                  _full((1, L)), _full((1, L)),
                  _full((L, L)), _full((L, L)), _full((1, L))],
        out_specs=[_rows(BN, L)] * 3,
        out_shape=[jax.ShapeDtypeStruct((N, L), _f32)] * 3,
    )(world_pos, prev_world_pos, node_type[:, None],
      params['node_mean'][None], params['node_std'][None],
      p['W1'], p['b1'][None], p['W2'], p['W3'], p['b2'][None], p['b3'][None],
      p['g'][None], p['be'][None],
      pe0['W1'][L:2 * L], pe0['W1'][2 * L:3 * L], pe0['b1'][None])


def _edge_encoder_body(ts_ref, td_ref, mean, std, w1, b1, w2, w3, b2, b3, g,
                       be, o_ref):
    d = ts_ref[:, 0:5] - td_ref[:, 0:5]
    dm = d[:, 0:2]
    dw = d[:, 2:5]
    nm = jnp.sqrt(jnp.sum(dm * dm, axis=-1, keepdims=True))
    nw = jnp.sqrt(jnp.sum(dw * dw, axis=-1, keepdims=True))
    f = jnp.concatenate([dm, nm, dw, nw], axis=-1)
    f = (f - mean[...]) / std[...]
    h = jnp.maximum(_dot(f, w1[...]) + b1[...], 0.0)
    h = jnp.maximum(_dot(h, w2[...]) + b2[...], 0.0)
    y = _dot(h, w3[...]) + b3[...]
    o_ref[...] = _ln(y, g[...], be[...])


def _edge_encoder(ts, td, params):
    p = params['edge_enc']
    return pl.pallas_call(
        _edge_encoder_body,
        grid=(E_PAD // BE,),
        in_specs=[_rows(BE, L), _rows(BE, L),
                  _full((1, 7)), _full((1, 7)),
                  _full((7, L)), _full((1, L)),
                  _full((L, L)), _full((L, L)), _full((1, L)), _full((1, L)),
                  _full((1, L)), _full((1, L))],
        out_specs=_rows(BE, L),
        out_shape=jax.ShapeDtypeStruct((E_PAD, L), _f32),
    )(ts, td, params['edge_mean'][None], params['edge_std'][None],
      p['W1'], p['b1'][None], p['W2'], p['W3'], p['b2'][None], p['b3'][None],
      p['g'][None], p['be'][None])


def _decoder_body(x_ref, w1, b1, w2, b2, w3, b3, ostd, omean, o_ref):
    h = jnp.maximum(_dot(x_ref[...], w1[...]) + b1[...], 0.0)
    h = jnp.maximum(_dot(h, w2[...]) + b2[...], 0.0)
    y = _dot(h, w3[...]) + b3[...]
    o_ref[...] = y * ostd[...] + omean[...]


def _decoder(nl, params):
    p = params['decoder']
    w3p = jnp.zeros((L, L), _f32).at[:, 0:3].set(p['W3'])
    b3p = jnp.zeros((1, L), _f32).at[:, 0:3].set(p['b3'][None])
    ostd = jnp.ones((1, L), _f32).at[:, 0:3].set(params['out_std'][None])
    omean = jnp.zeros((1, L), _f32).at[:, 0:3].set(params['out_mean'][None])
    out = pl.pallas_call(
        _decoder_body,
        grid=(N // BN,),
        in_specs=[_rows(BN, L),
                  _full((L, L)), _full((1, L)), _full((L, L)), _full((1, L)),
                  _full((L, L)), _full((1, L)), _full((1, L)), _full((1, L))],
        out_specs=_rows(BN, L),
        out_shape=jax.ShapeDtypeStruct((N, L), _f32),
    )(nl, p['W1'], p['b1'][None], p['W2'], p['b2'][None], w3p, b3p, ostd,
      omean)
    return out[:, 0:3]


def kernel(world_pos, prev_world_pos, target_world_pos, mesh_pos, node_type,
           cells, params):
    del target_world_pos
    a, b, c = cells[:, 0], cells[:, 1], cells[:, 2]
    srcs = jnp.concatenate([a, b, c, b, c, a]).astype(jnp.int32)
    dsts = jnp.concatenate([b, c, a, a, b, c]).astype(jnp.int32)
    pad = E_PAD - E
    si = jnp.pad(srcs, (0, pad)).reshape(NW, NCH, CHUNK)
    di_g = jnp.pad(dsts, (0, pad)).reshape(NW, NCH, CHUNK)
    di_s = jnp.pad(dsts, (0, pad), constant_values=JUNK).reshape(NW, NCH, CHUNK)

    tbl = jnp.concatenate(
        [mesh_pos, world_pos, jnp.zeros((N, L - 5), _f32)], axis=1)
    ts, td = _make_sc_gather_pair(L)(tbl, tbl, si, di_g)
    el = _edge_encoder(ts, td, params)
    nl, ps, pd = _node_encoder(world_pos, prev_world_pos, node_type, params)

    zero = jnp.zeros((ROWS_PER_TILE, L), _f32)
    gather = _make_sc_gather_pair(L)
    scatter = _make_sc_scatter_add()
    for i in range(STEPS):
        gs, gd = gather(ps, pd, si, di_g)
        el = _edge_update(el, gs, gd, params['proc_edge'][i])
        agg = scatter(el, di_s, zero)
        pe_next = params['proc_edge'][i + 1] if i + 1 < STEPS else None
        nl, ps, pd = _node_update(nl, agg[0, :N], agg[1, :N],
                                  params['proc_node'][i], pe_next)
    return _decoder(nl, params)


# speedup vs baseline: 3.1769x; 1.4771x over previous
import functools

import jax
import jax.numpy as jnp
from jax import lax
from jax.experimental import pallas as pl
from jax.experimental.pallas import tpu as pltpu
from jax.experimental.pallas import tpu_sc as plsc

N = 10000
C = 20000
E = 6 * C
L = 128
STEPS = 15
NC, NS = 2, 16
NW = NC * NS
CHUNK = 96
CPG = 20480
G_ROWS = 3 * CPG
NCHK = G_ROWS // (NW * CHUNK)
AGG_ROWS = 10240
ROWS_PER_TILE = AGG_ROWS // NS
JUNK = AGG_ROWS - 1

_f32 = jnp.float32

SMAP = (0, 1, 2, 1, 2, 0)
DMAP = (1, 2, 0, 0, 1, 2)
BC = 1000
NBC = C // BC


def _sc_mesh():
    return plsc.VectorSubcoreMesh(core_axis_name="c", subcore_axis_name="s")


@functools.lru_cache(maxsize=None)
def _make_sc_gather_cells(width):
    @functools.partial(
        pl.kernel,
        out_type=jax.ShapeDtypeStruct((G_ROWS, width), _f32),
        mesh=_sc_mesh(),
        scratch_types=(
            [pltpu.VMEM((NCHK, CHUNK), jnp.int32)]
            + [pltpu.VMEM((CHUNK, width), _f32)] * 4
            + [pltpu.SemaphoreType.DMA] * 8
        ),
    )
    def gather_cells(tbl_hbm, idx_hbm, out_hbm, idx_v, *bufs_sems):
        buf = bufs_sems[0:4]
        sg = bufs_sems[4:8]
        sw = bufs_sems[8:12]
        wid = lax.axis_index("s") * NC + lax.axis_index("c")
        base = wid * (NCHK * CHUNK)
        pltpu.sync_copy(idx_hbm.at[wid], idx_v)

        def g_start(slot, v):
            pltpu.async_copy(tbl_hbm.at[idx_v.at[v]], buf[slot], sg[slot])

        def g_wait(slot):
            pltpu.make_async_copy(tbl_hbm.at[idx_v.at[0]], buf[slot],
                                  sg[slot]).wait()

        def w_start(slot, v):
            pltpu.async_copy(buf[slot],
                             out_hbm.at[pl.ds(base + v * CHUNK, CHUNK)],
                             sw[slot])

        def w_wait(slot):
            pltpu.make_async_copy(buf[slot], out_hbm.at[pl.ds(base, CHUNK)],
                                  sw[slot]).wait()

        g_start(0, 0)
        g_start(1, 1)
        g_wait(0)
        w_start(0, 0)
        g_start(2, 2)
        g_wait(1)
        w_start(1, 1)
        g_start(3, 3)

        def body(i, carry):
            for b in range(4):
                v = 2 + i * 4 + b
                slot = (2 + b) % 4
                slot2 = (slot + 2) % 4
                g_wait(slot)
                w_start(slot, v)
                w_wait(slot2)
                g_start(slot2, v + 2)
            return carry

        lax.fori_loop(0, (NCHK - 4) // 4, body, 0)
        g_wait((NCHK - 2) % 4)
        w_start((NCHK - 2) % 4, NCHK - 2)
        g_wait((NCHK - 1) % 4)
        w_start((NCHK - 1) % 4, NCHK - 1)
        for slot in range(4):
            w_wait(slot)

    return gather_cells


@functools.lru_cache(maxsize=None)
def _make_sc_scatter_add():
    @functools.partial(
        pl.kernel,
        out_type=jax.ShapeDtypeStruct((NC, AGG_ROWS, L), _f32),
        mesh=_sc_mesh(),
        scratch_types=(
            [pltpu.VMEM((NCHK, CHUNK), jnp.int32)]
            + [pltpu.VMEM((CHUNK, L), _f32)] * 2
            + [pltpu.SemaphoreType.DMA] * 4
            + [pltpu.VMEM_SHARED((AGG_ROWS, L), _f32)]
        ),
    )
    def scatter_add(m_hbm, di_hbm, zero_hbm, agg_hbm, di_v, *bufs_sems):
        buf = bufs_sems[0:2]
        sl = bufs_sems[2:4]
        sa = bufs_sems[4:6]
        acc_sh = bufs_sems[6]
        c = lax.axis_index("c")
        s = lax.axis_index("s")
        wid = s * NC + c
        base = wid * (NCHK * CHUNK)
        row0 = s * ROWS_PER_TILE
        pltpu.sync_copy(di_hbm.at[wid], di_v)
        pltpu.sync_copy(zero_hbm, acc_sh.at[pl.ds(row0, ROWS_PER_TILE)])
        plsc.subcore_barrier()

        def l_start(slot, v):
            pltpu.async_copy(m_hbm.at[pl.ds(base + v * CHUNK, CHUNK)],
                             buf[slot], sl[slot])

        def l_wait(slot):
            pltpu.make_async_copy(m_hbm.at[pl.ds(base, CHUNK)], buf[slot],
                                  sl[slot]).wait()

        def a_start(slot, v):
            pltpu.async_copy(buf[slot], acc_sh.at[di_v.at[v]], sa[slot],
                             add=True)

        def a_wait(slot):
            pltpu.make_async_copy(buf[slot], acc_sh.at[di_v.at[0]],
                                  sa[slot]).wait()

        l_start(0, 0)
        l_start(1, 1)
        l_wait(0)
        a_start(0, 0)

        def body(i, carry):
            for b_off in range(2):
                v = 1 + i * 2 + b_off
                slot = (1 + b_off) % 2
                other = 1 - slot
                l_wait(slot)
                a_start(slot, v)
                a_wait(other)
                l_start(other, v + 1)
            return carry

        lax.fori_loop(0, (NCHK - 2) // 2, body, 0)
        l_wait((NCHK - 1) % 2)
        a_start((NCHK - 1) % 2, NCHK - 1)
        a_wait((NCHK - 2) % 2)
        a_wait((NCHK - 1) % 2)
        plsc.subcore_barrier()
        pltpu.sync_copy(acc_sh.at[pl.ds(row0, ROWS_PER_TILE)],
                        agg_hbm.at[c, pl.ds(row0, ROWS_PER_TILE)])

    return scatter_add


def _dot(a, b):
    return jnp.dot(a, b, preferred_element_type=_f32)


def _ln(y, g, b):
    m = jnp.mean(y, axis=-1, keepdims=True)
    v = jnp.mean((y - m) ** 2, axis=-1, keepdims=True)
    return (y - m) * lax.rsqrt(v + 1e-5) * g + b


def _full(shape):
    return pl.BlockSpec(shape, lambda *_: (0,) * len(shape))


BN = 1000


def _eblock(i, j):
    return (j * NBC + i, 0)


def _edge_update_body(x_ref, gabc_ref, w1, w2, w3, b2, b3, g, be, o_ref,
                      m_ref):
    j = pl.program_id(1)

    def make_branch(jj):
        def br():
            x = x_ref[...]
            gsum = (gabc_ref[SMAP[jj], :, 0:L]
                    + gabc_ref[DMAP[jj], :, L:2 * L])
            h = jnp.maximum(_dot(x, w1[...]) + gsum, 0.0)
            h = jnp.maximum(_dot(h, w2[...]) + b2[...], 0.0)
            y = _dot(h, w3[...]) + b3[...]
            o = x + _ln(y, g[...], be[...])
            o_ref[...] = o
            if jj < 3:
                m_ref[DMAP[jj]] = o
            else:
                m_ref[DMAP[jj]] += o
        return br

    lax.switch(j, [make_branch(jj) for jj in range(6)])


def _edge_update(el, gabc, p):
    w1e = p['W1'][0:L]
    return pl.pallas_call(
        _edge_update_body,
        grid=(NBC, 6),
        in_specs=[pl.BlockSpec((BC, L), _eblock),
                  pl.BlockSpec((3, BC, 2 * L), lambda i, j: (0, i, 0)),
                  _full((L, L)), _full((L, L)), _full((L, L)),
                  _full((1, L)), _full((1, L)), _full((1, L)), _full((1, L))],
        out_specs=[pl.BlockSpec((BC, L), _eblock),
                   pl.BlockSpec((3, BC, L), lambda i, j: (0, i, 0))],
        out_shape=[jax.ShapeDtypeStruct((E, L), _f32),
                   jax.ShapeDtypeStruct((3, CPG, L), _f32)],
    )(el, gabc, w1e, p['W2'], p['W3'], p['b2'][None], p['b3'][None],
      p['g'][None], p['be'][None])


def _node_update_body(x_ref, a0_ref, a1_ref, w1n, w1a, b1, w2, w3, b2, b3, g,
                      be, ws, wd, bd, o_ref, ppd_ref, *, proj):
    x = x_ref[...]
    a = a0_ref[...] + a1_ref[...]
    h = jnp.maximum(_dot(x, w1n[...]) + _dot(a, w1a[...]) + b1[...], 0.0)
    h = jnp.maximum(_dot(h, w2[...]) + b2[...], 0.0)
    y = _dot(h, w3[...]) + b3[...]
    o = x + _ln(y, g[...], be[...])
    o_ref[...] = o
    if proj:
        ppd_ref[:, 0:L] = _dot(o, ws[...])
        ppd_ref[:, L:2 * L] = _dot(o, wd[...]) + bd[...]


def _node_update(nl, agg0, agg1, p, pe_next):
    proj = pe_next is not None
    w1n = p['W1'][0:L]
    w1a = p['W1'][L:2 * L]
    if proj:
        ws = pe_next['W1'][L:2 * L]
        wd = pe_next['W1'][2 * L:3 * L]
        bd = pe_next['b1'][None]
    else:
        ws = jnp.zeros((1, L), _f32)
        wd = jnp.zeros((1, L), _f32)
        bd = jnp.zeros((1, L), _f32)
    out_shapes = [jax.ShapeDtypeStruct((N, L), _f32),
                  jax.ShapeDtypeStruct((N, 2 * L), _f32)]
    out_specs = [pl.BlockSpec((BN, L), lambda i: (i, 0)),
                 pl.BlockSpec((BN, 2 * L), lambda i: (i, 0))]

    def body(*refs):
        if proj:
            _node_update_body(*refs, proj=True)
        else:
            _node_update_body(*refs[:16], None, proj=False)

    res = pl.pallas_call(
        body,
        grid=(N // BN,),
        in_specs=[pl.BlockSpec((BN, L), lambda i: (i, 0))] * 3 +
                 [_full((L, L)), _full((L, L)), _full((1, L)),
                  _full((L, L)), _full((L, L)), _full((1, L)), _full((1, L)),
                  _full((1, L)), _full((1, L)),
                  _full(ws.shape), _full(wd.shape), _full((1, L))],
        out_specs=out_specs if proj else out_specs[0],
        out_shape=out_shapes if proj else out_shapes[0],
    )(nl, agg0, agg1, w1n, w1a, p['b1'][None], p['W2'], p['W3'],
      p['b2'][None], p['b3'][None], p['g'][None], p['be'][None], ws, wd, bd)
    if proj:
        return res
    return res, None


def _node_encoder_body(wp_ref, pwp_ref, nt_ref, mean, std, w1, b1, w2, w3, b2,
                       b3, g, be, ws, wd, bd, o_ref, ppd_ref):
    vel = wp_ref[...] - pwp_ref[...]
    t = nt_ref[...]
    oh = (t == lax.broadcasted_iota(jnp.int32, (1, 9), 1)).astype(_f32)
    f = jnp.concatenate([vel, oh], axis=-1)
    f = (f - mean[...]) / std[...]
    h = jnp.maximum(_dot(f, w1[...]) + b1[...], 0.0)
    h = jnp.maximum(_dot(h, w2[...]) + b2[...], 0.0)
    y = _dot(h, w3[...]) + b3[...]
    o = _ln(y, g[...], be[...])
    o_ref[...] = o
    ppd_ref[:, 0:L] = _dot(o, ws[...])
    ppd_ref[:, L:2 * L] = _dot(o, wd[...]) + bd[...]


def _node_encoder(world_pos, prev_world_pos, node_type, params):
    p = params['node_enc']
    pe0 = params['proc_edge'][0]
    return pl.pallas_call(
        _node_encoder_body,
        grid=(N // BN,),
        in_specs=[pl.BlockSpec((BN, 3), lambda i: (i, 0)),
                  pl.BlockSpec((BN, 3), lambda i: (i, 0)),
                  pl.BlockSpec((BN, 1), lambda i: (i, 0)),
                  _full((1, 12)), _full((1, 12)),
                  _full((12, L)), _full((1, L)),
                  _full((L, L)), _full((L, L)), _full((1, L)), _full((1, L)),
                  _full((1, L)), _full((1, L)),
                  _full((L, L)), _full((L, L)), _full((1, L))],
        out_specs=[pl.BlockSpec((BN, L), lambda i: (i, 0)),
                   pl.BlockSpec((BN, 2 * L), lambda i: (i, 0))],
        out_shape=[jax.ShapeDtypeStruct((N, L), _f32),
                   jax.ShapeDtypeStruct((N, 2 * L), _f32)],
    )(world_pos, prev_world_pos, node_type[:, None],
      params['node_mean'][None], params['node_std'][None],
      p['W1'], p['b1'][None], p['W2'], p['W3'], p['b2'][None], p['b3'][None],
      p['g'][None], p['be'][None],
      pe0['W1'][L:2 * L], pe0['W1'][2 * L:3 * L], pe0['b1'][None])


def _edge_encoder_body(gt_ref, mean, std, w1, b1, w2, w3, b2, b3, g, be,
                       o_ref):
    j = pl.program_id(1)

    def make_branch(jj):
        def br():
            d = gt_ref[SMAP[jj], :, 0:5] - gt_ref[DMAP[jj], :, 0:5]
            dm = d[:, 0:2]
            dw = d[:, 2:5]
            nm = jnp.sqrt(jnp.sum(dm * dm, axis=-1, keepdims=True))
            nw = jnp.sqrt(jnp.sum(dw * dw, axis=-1, keepdims=True))
            f = jnp.concatenate([dm, nm, dw, nw], axis=-1)
            f = (f - mean[...]) / std[...]
            h = jnp.maximum(_dot(f, w1[...]) + b1[...], 0.0)
            h = jnp.maximum(_dot(h, w2[...]) + b2[...], 0.0)
            y = _dot(h, w3[...]) + b3[...]
            o_ref[...] = _ln(y, g[...], be[...])
        return br

    lax.switch(j, [make_branch(jj) for jj in range(6)])


def _edge_encoder(gt, params):
    p = params['edge_enc']
    return pl.pallas_call(
        _edge_encoder_body,
        grid=(NBC, 6),
        in_specs=[pl.BlockSpec((3, BC, L), lambda i, j: (0, i, 0)),
                  _full((1, 7)), _full((1, 7)),
                  _full((7, L)), _full((1, L)),
                  _full((L, L)), _full((L, L)), _full((1, L)), _full((1, L)),
                  _full((1, L)), _full((1, L))],
        out_specs=pl.BlockSpec((BC, L), _eblock),
        out_shape=jax.ShapeDtypeStruct((E, L), _f32),
    )(gt, params['edge_mean'][None], params['edge_std'][None],
      p['W1'], p['b1'][None], p['W2'], p['W3'], p['b2'][None], p['b3'][None],
      p['g'][None], p['be'][None])


def _decoder_body(x_ref, w1, b1, w2, b2, w3, b3, ostd, omean, o_ref):
    h = jnp.maximum(_dot(x_ref[...], w1[...]) + b1[...], 0.0)
    h = jnp.maximum(_dot(h, w2[...]) + b2[...], 0.0)
    y = _dot(h, w3[...]) + b3[...]
    o_ref[...] = y * ostd[...] + omean[...]


def _decoder(nl, params):
    p = params['decoder']
    w3p = jnp.zeros((L, L), _f32).at[:, 0:3].set(p['W3'])
    b3p = jnp.zeros((1, L), _f32).at[:, 0:3].set(p['b3'][None])
    ostd = jnp.ones((1, L), _f32).at[:, 0:3].set(params['out_std'][None])
    omean = jnp.zeros((1, L), _f32).at[:, 0:3].set(params['out_mean'][None])
    out = pl.pallas_call(
        _decoder_body,
        grid=(N // BN,),
        in_specs=[pl.BlockSpec((BN, L), lambda i: (i, 0)),
                  _full((L, L)), _full((1, L)), _full((L, L)), _full((1, L)),
                  _full((L, L)), _full((1, L)), _full((1, L)), _full((1, L))],
        out_specs=pl.BlockSpec((BN, L), lambda i: (i, 0)),
        out_shape=jax.ShapeDtypeStruct((N, L), _f32),
    )(nl, p['W1'], p['b1'][None], p['W2'], p['b2'][None], w3p, b3p, ostd,
      omean)
    return out[:, 0:3]


def _cells_idx(cells, pad_value):
    cols = []
    for k in range(3):
        cols.append(jnp.pad(cells[:, k].astype(jnp.int32), (0, CPG - C),
                            constant_values=pad_value))
    return jnp.concatenate(cols).reshape(NW, NCHK, CHUNK)


def kernel(world_pos, prev_world_pos, target_world_pos, mesh_pos, node_type,
           cells, params):
    del target_world_pos
    gi = _cells_idx(cells, 0)
    si = _cells_idx(cells, JUNK)

    tbl = jnp.concatenate(
        [mesh_pos, world_pos, jnp.zeros((N, L - 5), _f32)], axis=1)
    gt = _make_sc_gather_cells(L)(tbl, gi).reshape(3, CPG, L)
    el = _edge_encoder(gt, params)
    nl, ppd = _node_encoder(world_pos, prev_world_pos, node_type, params)

    zero = jnp.zeros((ROWS_PER_TILE, L), _f32)
    gather = _make_sc_gather_cells(2 * L)
    scatter = _make_sc_scatter_add()
    for i in range(STEPS):
        gabc = gather(ppd, gi).reshape(3, CPG, 2 * L)
        el, m = _edge_update(el, gabc, params['proc_edge'][i])
        agg = scatter(m.reshape(G_ROWS, L), si, zero)
        pe_next = params['proc_edge'][i + 1] if i + 1 < STEPS else None
        nl, ppd = _node_update(nl, agg[0, :N], agg[1, :N],
                               params['proc_node'][i], pe_next)
    return _decoder(nl, params)


# speedup vs baseline: 3.2258x; 1.0154x over previous
import functools

import jax
import jax.numpy as jnp
from jax import lax
from jax.experimental import pallas as pl
from jax.experimental.pallas import tpu as pltpu
from jax.experimental.pallas import tpu_sc as plsc

N = 10000
C = 20000
E = 6 * C
L = 128
STEPS = 15
NC, NS = 2, 16
NW = NC * NS
CPG = 20480
G_ROWS = 3 * CPG
CHUNK_G = 128
NCHK_G = G_ROWS // (NW * CHUNK_G)
CHUNK_S = 64
NCHK_S = G_ROWS // (NW * CHUNK_S)
AGG_ROWS = 10240
ROWS_PER_TILE = AGG_ROWS // NS
JUNK = AGG_ROWS - 1

_f32 = jnp.float32

SMAP = (0, 1, 2, 1, 2, 0)
DMAP = (1, 2, 0, 0, 1, 2)
BC = 1000
NBC = C // BC


def _sc_mesh():
    return plsc.VectorSubcoreMesh(core_axis_name="c", subcore_axis_name="s")


@functools.lru_cache(maxsize=None)
def _make_sc_gather_cells(width):
    assert (NCHK_G - 3) % 3 == 0

    @functools.partial(
        pl.kernel,
        out_type=jax.ShapeDtypeStruct((G_ROWS, width), _f32),
        mesh=_sc_mesh(),
        scratch_types=(
            [pltpu.VMEM((NCHK_G, CHUNK_G), jnp.int32)]
            + [pltpu.VMEM((CHUNK_G, width), _f32)] * 3
            + [pltpu.SemaphoreType.DMA] * 6
        ),
    )
    def gather_cells(tbl_hbm, idx_hbm, out_hbm, idx_v, *bufs_sems):
        buf = bufs_sems[0:3]
        sg = bufs_sems[3:6]
        sw = bufs_sems[6:9]
        wid = lax.axis_index("s") * NC + lax.axis_index("c")
        base = wid * (NCHK_G * CHUNK_G)
        pltpu.sync_copy(idx_hbm.at[wid], idx_v)

        def g_start(slot, v):
            pltpu.async_copy(tbl_hbm.at[idx_v.at[v]], buf[slot], sg[slot])

        def g_wait(slot):
            pltpu.make_async_copy(tbl_hbm.at[idx_v.at[0]], buf[slot],
                                  sg[slot]).wait()

        def w_start(slot, v):
            pltpu.async_copy(buf[slot],
                             out_hbm.at[pl.ds(base + v * CHUNK_G, CHUNK_G)],
                             sw[slot])

        def w_wait(slot):
            pltpu.make_async_copy(buf[slot], out_hbm.at[pl.ds(base, CHUNK_G)],
                                  sw[slot]).wait()

        g_start(0, 0)
        g_start(1, 1)
        g_wait(0)
        w_start(0, 0)
        g_start(2, 2)

        def body(i, carry):
            for b in range(3):
                v = 1 + i * 3 + b
                slot = (1 + b) % 3
                slot2 = b % 3
                g_wait(slot)
                w_start(slot, v)
                w_wait(slot2)
                g_start(slot2, v + 2)
            return carry

        lax.fori_loop(0, (NCHK_G - 3) // 3, body, 0)
        g_wait((NCHK_G - 2) % 3)
        w_start((NCHK_G - 2) % 3, NCHK_G - 2)
        w_wait((NCHK_G - 3) % 3)
        g_wait((NCHK_G - 1) % 3)
        w_start((NCHK_G - 1) % 3, NCHK_G - 1)
        w_wait((NCHK_G - 2) % 3)
        w_wait((NCHK_G - 1) % 3)

    return gather_cells


@functools.lru_cache(maxsize=None)
def _make_sc_scatter_add():
    assert (NCHK_S - 5) % 5 == 0

    @functools.partial(
        pl.kernel,
        out_type=jax.ShapeDtypeStruct((NC, AGG_ROWS, L), _f32),
        mesh=_sc_mesh(),
        scratch_types=(
            [pltpu.VMEM((NCHK_S, CHUNK_S), jnp.int32)]
            + [pltpu.VMEM((CHUNK_S, L), _f32)] * 5
            + [pltpu.SemaphoreType.DMA] * 10
            + [pltpu.VMEM_SHARED((AGG_ROWS, L), _f32)]
        ),
    )
    def scatter_add(m_hbm, di_hbm, zero_hbm, agg_hbm, di_v, *bufs_sems):
        buf = bufs_sems[0:5]
        sl = bufs_sems[5:10]
        sa = bufs_sems[10:15]
        acc_sh = bufs_sems[15]
        c = lax.axis_index("c")
        s = lax.axis_index("s")
        wid = s * NC + c
        base = wid * (NCHK_S * CHUNK_S)
        row0 = s * ROWS_PER_TILE
        pltpu.sync_copy(di_hbm.at[wid], di_v)
        pltpu.sync_copy(zero_hbm, acc_sh.at[pl.ds(row0, ROWS_PER_TILE)])
        plsc.subcore_barrier()

        def l_start(slot, v):
            pltpu.async_copy(m_hbm.at[pl.ds(base + v * CHUNK_S, CHUNK_S)],
                             buf[slot], sl[slot])

        def l_wait(slot):
            pltpu.make_async_copy(m_hbm.at[pl.ds(base, CHUNK_S)], buf[slot],
                                  sl[slot]).wait()

        def a_start(slot, v):
            pltpu.async_copy(buf[slot], acc_sh.at[di_v.at[v]], sa[slot],
                             add=True)

        def a_wait(slot):
            pltpu.make_async_copy(buf[slot], acc_sh.at[di_v.at[0]],
                                  sa[slot]).wait()

        l_start(0, 0)
        l_start(1, 1)
        for v0 in range(3):
            l_wait(v0)
            a_start(v0, v0)
            l_start(v0 + 2, v0 + 2)

        def body(i, carry):
            for b in range(5):
                v = 3 + i * 5 + b
                slot = (3 + b) % 5
                slot2 = b % 5
                l_wait(slot)
                a_start(slot, v)
                a_wait(slot2)
                l_start(slot2, v + 2)
            return carry

        lax.fori_loop(0, (NCHK_S - 5) // 5, body, 0)
        l_wait((NCHK_S - 2) % 5)
        a_start((NCHK_S - 2) % 5, NCHK_S - 2)
        a_wait((NCHK_S - 5) % 5)
        l_wait((NCHK_S - 1) % 5)
        a_start((NCHK_S - 1) % 5, NCHK_S - 1)
        a_wait((NCHK_S - 4) % 5)
        a_wait((NCHK_S - 3) % 5)
        a_wait((NCHK_S - 2) % 5)
        a_wait((NCHK_S - 1) % 5)
        plsc.subcore_barrier()
        pltpu.sync_copy(acc_sh.at[pl.ds(row0, ROWS_PER_TILE)],
                        agg_hbm.at[c, pl.ds(row0, ROWS_PER_TILE)])

    return scatter_add


def _dot(a, b):
    return jnp.dot(a, b, preferred_element_type=_f32)


def _ln(y, g, b):
    m = jnp.mean(y, axis=-1, keepdims=True)
    v = jnp.mean((y - m) ** 2, axis=-1, keepdims=True)
    return (y - m) * lax.rsqrt(v + 1e-5) * g + b


def _full(shape):
    return pl.BlockSpec(shape, lambda *_: (0,) * len(shape))


BN = 1000


def _eblock(i, j):
    return (j * NBC + i, 0)


def _edge_update_body(x_ref, gabc_ref, w1, w2, w3, b2, b3, g, be, o_ref,
                      m_ref):
    j = pl.program_id(1)

    def make_branch(jj):
        def br():
            x = x_ref[...]
            gsum = (gabc_ref[SMAP[jj], :, 0:L]
                    + gabc_ref[DMAP[jj], :, L:2 * L])
            h = jnp.maximum(_dot(x, w1[...]) + gsum, 0.0)
            h = jnp.maximum(_dot(h, w2[...]) + b2[...], 0.0)
            y = _dot(h, w3[...]) + b3[...]
            o = x + _ln(y, g[...], be[...])
            o_ref[...] = o
            if jj < 3:
                m_ref[DMAP[jj]] = o
            else:
                m_ref[DMAP[jj]] += o
        return br

    lax.switch(j, [make_branch(jj) for jj in range(6)])


def _edge_update(el, gabc, p):
    w1e = p['W1'][0:L]
    return pl.pallas_call(
        _edge_update_body,
        grid=(NBC, 6),
        in_specs=[pl.BlockSpec((BC, L), _eblock),
                  pl.BlockSpec((3, BC, 2 * L), lambda i, j: (0, i, 0)),
                  _full((L, L)), _full((L, L)), _full((L, L)),
                  _full((1, L)), _full((1, L)), _full((1, L)), _full((1, L))],
        out_specs=[pl.BlockSpec((BC, L), _eblock),
                   pl.BlockSpec((3, BC, L), lambda i, j: (0, i, 0))],
        out_shape=[jax.ShapeDtypeStruct((E, L), _f32),
                   jax.ShapeDtypeStruct((3, CPG, L), _f32)],
    )(el, gabc, w1e, p['W2'], p['W3'], p['b2'][None], p['b3'][None],
      p['g'][None], p['be'][None])


def _node_update_body(x_ref, a0_ref, a1_ref, w1n, w1a, b1, w2, w3, b2, b3, g,
                      be, ws, wd, bd, o_ref, ppd_ref, *, proj):
    x = x_ref[...]
    a = a0_ref[...] + a1_ref[...]
    h = jnp.maximum(_dot(x, w1n[...]) + _dot(a, w1a[...]) + b1[...], 0.0)
    h = jnp.maximum(_dot(h, w2[...]) + b2[...], 0.0)
    y = _dot(h, w3[...]) + b3[...]
    o = x + _ln(y, g[...], be[...])
    o_ref[...] = o
    if proj:
        ppd_ref[:, 0:L] = _dot(o, ws[...])
        ppd_ref[:, L:2 * L] = _dot(o, wd[...]) + bd[...]


def _node_update(nl, agg0, agg1, p, pe_next):
    proj = pe_next is not None
    w1n = p['W1'][0:L]
    w1a = p['W1'][L:2 * L]
    if proj:
        ws = pe_next['W1'][L:2 * L]
        wd = pe_next['W1'][2 * L:3 * L]
        bd = pe_next['b1'][None]
    else:
        ws = jnp.zeros((1, L), _f32)
        wd = jnp.zeros((1, L), _f32)
        bd = jnp.zeros((1, L), _f32)
    out_shapes = [jax.ShapeDtypeStruct((N, L), _f32),
                  jax.ShapeDtypeStruct((N, 2 * L), _f32)]
    out_specs = [pl.BlockSpec((BN, L), lambda i: (i, 0)),
                 pl.BlockSpec((BN, 2 * L), lambda i: (i, 0))]

    def body(*refs):
        if proj:
            _node_update_body(*refs, proj=True)
        else:
            _node_update_body(*refs[:16], None, proj=False)

    res = pl.pallas_call(
        body,
        grid=(N // BN,),
        in_specs=[pl.BlockSpec((BN, L), lambda i: (i, 0))] * 3 +
                 [_full((L, L)), _full((L, L)), _full((1, L)),
                  _full((L, L)), _full((L, L)), _full((1, L)), _full((1, L)),
                  _full((1, L)), _full((1, L)),
                  _full(ws.shape), _full(wd.shape), _full((1, L))],
        out_specs=out_specs if proj else out_specs[0],
        out_shape=out_shapes if proj else out_shapes[0],
    )(nl, agg0, agg1, w1n, w1a, p['b1'][None], p['W2'], p['W3'],
      p['b2'][None], p['b3'][None], p['g'][None], p['be'][None], ws, wd, bd)
    if proj:
        return res
    return res, None


def _node_encoder_body(wp_ref, pwp_ref, nt_ref, mean, std, w1, b1, w2, w3, b2,
                       b3, g, be, ws, wd, bd, o_ref, ppd_ref):
    vel = wp_ref[...] - pwp_ref[...]
    t = nt_ref[...]
    oh = (t == lax.broadcasted_iota(jnp.int32, (1, 9), 1)).astype(_f32)
    f = jnp.concatenate([vel, oh], axis=-1)
    f = (f - mean[...]) / std[...]
    h = jnp.maximum(_dot(f, w1[...]) + b1[...], 0.0)
    h = jnp.maximum(_dot(h, w2[...]) + b2[...], 0.0)
    y = _dot(h, w3[...]) + b3[...]
    o = _ln(y, g[...], be[...])
    o_ref[...] = o
    ppd_ref[:, 0:L] = _dot(o, ws[...])
    ppd_ref[:, L:2 * L] = _dot(o, wd[...]) + bd[...]


def _node_encoder(world_pos, prev_world_pos, node_type, params):
    p = params['node_enc']
    pe0 = params['proc_edge'][0]
    return pl.pallas_call(
        _node_encoder_body,
        grid=(N // BN,),
        in_specs=[pl.BlockSpec((BN, 3), lambda i: (i, 0)),
                  pl.BlockSpec((BN, 3), lambda i: (i, 0)),
                  pl.BlockSpec((BN, 1), lambda i: (i, 0)),
                  _full((1, 12)), _full((1, 12)),
                  _full((12, L)), _full((1, L)),
                  _full((L, L)), _full((L, L)), _full((1, L)), _full((1, L)),
                  _full((1, L)), _full((1, L)),
                  _full((L, L)), _full((L, L)), _full((1, L))],
        out_specs=[pl.BlockSpec((BN, L), lambda i: (i, 0)),
                   pl.BlockSpec((BN, 2 * L), lambda i: (i, 0))],
        out_shape=[jax.ShapeDtypeStruct((N, L), _f32),
                   jax.ShapeDtypeStruct((N, 2 * L), _f32)],
    )(world_pos, prev_world_pos, node_type[:, None],
      params['node_mean'][None], params['node_std'][None],
      p['W1'], p['b1'][None], p['W2'], p['W3'], p['b2'][None], p['b3'][None],
      p['g'][None], p['be'][None],
      pe0['W1'][L:2 * L], pe0['W1'][2 * L:3 * L], pe0['b1'][None])


def _edge_encoder_body(gt_ref, mean, std, w1, b1, w2, w3, b2, b3, g, be,
                       o_ref):
    j = pl.program_id(1)

    def make_branch(jj):
        def br():
            d = gt_ref[SMAP[jj], :, 0:5] - gt_ref[DMAP[jj], :, 0:5]
            dm = d[:, 0:2]
            dw = d[:, 2:5]
            nm = jnp.sqrt(jnp.sum(dm * dm, axis=-1, keepdims=True))
            nw = jnp.sqrt(jnp.sum(dw * dw, axis=-1, keepdims=True))
            f = jnp.concatenate([dm, nm, dw, nw], axis=-1)
            f = (f - mean[...]) / std[...]
            h = jnp.maximum(_dot(f, w1[...]) + b1[...], 0.0)
            h = jnp.maximum(_dot(h, w2[...]) + b2[...], 0.0)
            y = _dot(h, w3[...]) + b3[...]
            o_ref[...] = _ln(y, g[...], be[...])
        return br

    lax.switch(j, [make_branch(jj) for jj in range(6)])


def _edge_encoder(gt, params):
    p = params['edge_enc']
    return pl.pallas_call(
        _edge_encoder_body,
        grid=(NBC, 6),
        in_specs=[pl.BlockSpec((3, BC, L), lambda i, j: (0, i, 0)),
                  _full((1, 7)), _full((1, 7)),
                  _full((7, L)), _full((1, L)),
                  _full((L, L)), _full((L, L)), _full((1, L)), _full((1, L)),
                  _full((1, L)), _full((1, L))],
        out_specs=pl.BlockSpec((BC, L), _eblock),
        out_shape=jax.ShapeDtypeStruct((E, L), _f32),
    )(gt, params['edge_mean'][None], params['edge_std'][None],
      p['W1'], p['b1'][None], p['W2'], p['W3'], p['b2'][None], p['b3'][None],
      p['g'][None], p['be'][None])


def _decoder_body(x_ref, w1, b1, w2, b2, w3, b3, ostd, omean, o_ref):
    h = jnp.maximum(_dot(x_ref[...], w1[...]) + b1[...], 0.0)
    h = jnp.maximum(_dot(h, w2[...]) + b2[...], 0.0)
    y = _dot(h, w3[...]) + b3[...]
    o_ref[...] = y * ostd[...] + omean[...]


def _decoder(nl, params):
    p = params['decoder']
    w3p = jnp.zeros((L, L), _f32).at[:, 0:3].set(p['W3'])
    b3p = jnp.zeros((1, L), _f32).at[:, 0:3].set(p['b3'][None])
    ostd = jnp.ones((1, L), _f32).at[:, 0:3].set(params['out_std'][None])
    omean = jnp.zeros((1, L), _f32).at[:, 0:3].set(params['out_mean'][None])
    out = pl.pallas_call(
        _decoder_body,
        grid=(N // BN,),
        in_specs=[pl.BlockSpec((BN, L), lambda i: (i, 0)),
                  _full((L, L)), _full((1, L)), _full((L, L)), _full((1, L)),
                  _full((L, L)), _full((1, L)), _full((1, L)), _full((1, L))],
        out_specs=pl.BlockSpec((BN, L), lambda i: (i, 0)),
        out_shape=jax.ShapeDtypeStruct((N, L), _f32),
    )(nl, p['W1'], p['b1'][None], p['W2'], p['b2'][None], w3p, b3p, ostd,
      omean)
    return out[:, 0:3]


def _cells_idx(cells, pad_value, nchk, chunk):
    cols = []
    for k in range(3):
        cols.append(jnp.pad(cells[:, k].astype(jnp.int32), (0, CPG - C),
                            constant_values=pad_value))
    return jnp.concatenate(cols).reshape(NW, nchk, chunk)


def kernel(world_pos, prev_world_pos, target_world_pos, mesh_pos, node_type,
           cells, params):
    del target_world_pos
    gi = _cells_idx(cells, 0, NCHK_G, CHUNK_G)
    si = _cells_idx(cells, JUNK, NCHK_S, CHUNK_S)

    tbl = jnp.concatenate(
        [mesh_pos, world_pos, jnp.zeros((N, L - 5), _f32)], axis=1)
    gt = _make_sc_gather_cells(L)(tbl, gi).reshape(3, CPG, L)
    el = _edge_encoder(gt, params)
    nl, ppd = _node_encoder(world_pos, prev_world_pos, node_type, params)

    zero = jnp.zeros((ROWS_PER_TILE, L), _f32)
    gather = _make_sc_gather_cells(2 * L)
    scatter = _make_sc_scatter_add()
    for i in range(STEPS):
        gabc = gather(ppd, gi).reshape(3, CPG, 2 * L)
        el, m = _edge_update(el, gabc, params['proc_edge'][i])
        agg = scatter(m.reshape(G_ROWS, L), si, zero)
        pe_next = params['proc_edge'][i + 1] if i + 1 < STEPS else None
        nl, ppd = _node_update(nl, agg[0, :N], agg[1, :N],
                               params['proc_node'][i], pe_next)
    return _decoder(nl, params)


# speedup vs baseline: 3.2767x; 1.0158x over previous
import functools

import jax
import jax.numpy as jnp
from jax import lax
from jax.experimental import pallas as pl
from jax.experimental.pallas import tpu as pltpu
from jax.experimental.pallas import tpu_sc as plsc

N = 10000
C = 20000
E = 6 * C
L = 128
STEPS = 15
NC, NS = 2, 16
NW = NC * NS
CPG = 20480
G_ROWS = 3 * CPG
CHUNK_G = 128
NCHK_G = G_ROWS // (NW * CHUNK_G)
CHUNK_S = 64
NCHK_S = G_ROWS // (NW * CHUNK_S)
AGG_ROWS = 10240
ROWS_PER_TILE = AGG_ROWS // NS
JUNK = AGG_ROWS - 1

_f32 = jnp.float32

SMAP = (0, 1, 2, 1, 2, 0)
DMAP = (1, 2, 0, 0, 1, 2)
BC = 1000
NBC = C // BC


def _sc_mesh():
    return plsc.VectorSubcoreMesh(core_axis_name="c", subcore_axis_name="s")


@functools.lru_cache(maxsize=None)
def _make_sc_gather_cells(width):
    assert (NCHK_G - 3) % 3 == 0

    @functools.partial(
        pl.kernel,
        out_type=jax.ShapeDtypeStruct((G_ROWS, width), _f32),
        mesh=_sc_mesh(),
        scratch_types=(
            [pltpu.VMEM((NCHK_G, CHUNK_G), jnp.int32)]
            + [pltpu.VMEM((CHUNK_G, width), _f32)] * 3
            + [pltpu.SemaphoreType.DMA] * 6
        ),
    )
    def gather_cells(tbl_hbm, idx_hbm, out_hbm, idx_v, *bufs_sems):
        buf = bufs_sems[0:3]
        sg = bufs_sems[3:6]
        sw = bufs_sems[6:9]
        wid = lax.axis_index("s") * NC + lax.axis_index("c")
        base = wid * (NCHK_G * CHUNK_G)
        pltpu.sync_copy(idx_hbm.at[wid], idx_v)

        def g_start(slot, v):
            pltpu.async_copy(tbl_hbm.at[idx_v.at[v]], buf[slot], sg[slot])

        def g_wait(slot):
            pltpu.make_async_copy(tbl_hbm.at[idx_v.at[0]], buf[slot],
                                  sg[slot]).wait()

        def w_start(slot, v):
            pltpu.async_copy(buf[slot],
                             out_hbm.at[pl.ds(base + v * CHUNK_G, CHUNK_G)],
                             sw[slot])

        def w_wait(slot):
            pltpu.make_async_copy(buf[slot], out_hbm.at[pl.ds(base, CHUNK_G)],
                                  sw[slot]).wait()

        g_start(0, 0)
        g_start(1, 1)
        g_wait(0)
        w_start(0, 0)
        g_start(2, 2)

        def body(i, carry):
            for b in range(3):
                v = 1 + i * 3 + b
                slot = (1 + b) % 3
                slot2 = b % 3
                g_wait(slot)
                w_start(slot, v)
                w_wait(slot2)
                g_start(slot2, v + 2)
            return carry

        lax.fori_loop(0, (NCHK_G - 3) // 3, body, 0)
        g_wait((NCHK_G - 2) % 3)
        w_start((NCHK_G - 2) % 3, NCHK_G - 2)
        w_wait((NCHK_G - 3) % 3)
        g_wait((NCHK_G - 1) % 3)
        w_start((NCHK_G - 1) % 3, NCHK_G - 1)
        w_wait((NCHK_G - 2) % 3)
        w_wait((NCHK_G - 1) % 3)

    return gather_cells


@functools.lru_cache(maxsize=None)
def _make_sc_scatter_add():
    assert (NCHK_S - 5) % 5 == 0

    @functools.partial(
        pl.kernel,
        out_type=jax.ShapeDtypeStruct((NC, AGG_ROWS, L), _f32),
        mesh=_sc_mesh(),
        scratch_types=(
            [pltpu.VMEM((NCHK_S, CHUNK_S), jnp.int32)]
            + [pltpu.VMEM((CHUNK_S, L), _f32)] * 5
            + [pltpu.SemaphoreType.DMA] * 10
            + [pltpu.VMEM_SHARED((AGG_ROWS, L), _f32)]
        ),
    )
    def scatter_add(m_hbm, di_hbm, zero_hbm, agg_hbm, di_v, *bufs_sems):
        buf = bufs_sems[0:5]
        sl = bufs_sems[5:10]
        sa = bufs_sems[10:15]
        acc_sh = bufs_sems[15]
        c = lax.axis_index("c")
        s = lax.axis_index("s")
        wid = s * NC + c
        base = wid * (NCHK_S * CHUNK_S)
        row0 = s * ROWS_PER_TILE
        pltpu.sync_copy(di_hbm.at[wid], di_v)
        pltpu.sync_copy(zero_hbm, acc_sh.at[pl.ds(row0, ROWS_PER_TILE)])
        plsc.subcore_barrier()

        def l_start(slot, v):
            pltpu.async_copy(m_hbm.at[pl.ds(base + v * CHUNK_S, CHUNK_S)],
                             buf[slot], sl[slot])

        def l_wait(slot):
            pltpu.make_async_copy(m_hbm.at[pl.ds(base, CHUNK_S)], buf[slot],
                                  sl[slot]).wait()

        def a_start(slot, v):
            pltpu.async_copy(buf[slot], acc_sh.at[di_v.at[v]], sa[slot],
                             add=True)

        def a_wait(slot):
            pltpu.make_async_copy(buf[slot], acc_sh.at[di_v.at[0]],
                                  sa[slot]).wait()

        l_start(0, 0)
        l_start(1, 1)
        for v0 in range(3):
            l_wait(v0)
            a_start(v0, v0)
            l_start(v0 + 2, v0 + 2)

        def body(i, carry):
            for b in range(5):
                v = 3 + i * 5 + b
                slot = (3 + b) % 5
                slot2 = b % 5
                l_wait(slot)
                a_start(slot, v)
                a_wait(slot2)
                l_start(slot2, v + 2)
            return carry

        lax.fori_loop(0, (NCHK_S - 5) // 5, body, 0)
        l_wait((NCHK_S - 2) % 5)
        a_start((NCHK_S - 2) % 5, NCHK_S - 2)
        a_wait((NCHK_S - 5) % 5)
        l_wait((NCHK_S - 1) % 5)
        a_start((NCHK_S - 1) % 5, NCHK_S - 1)
        a_wait((NCHK_S - 4) % 5)
        a_wait((NCHK_S - 3) % 5)
        a_wait((NCHK_S - 2) % 5)
        a_wait((NCHK_S - 1) % 5)
        plsc.subcore_barrier()
        pltpu.sync_copy(acc_sh.at[pl.ds(row0, ROWS_PER_TILE)],
                        agg_hbm.at[c, pl.ds(row0, ROWS_PER_TILE)])

    return scatter_add


def _dot(a, b):
    return jnp.dot(a, b, preferred_element_type=_f32)


def _ln(y, g, b):
    m = jnp.mean(y, axis=-1, keepdims=True)
    v = jnp.mean((y - m) ** 2, axis=-1, keepdims=True)
    return (y - m) * lax.rsqrt(v + 1e-5) * g + b


def _full(shape):
    return pl.BlockSpec(shape, lambda *_: (0,) * len(shape))


BN = 1000


def _eblock(i, j):
    return (j * NBC + i, 0)


def _edge_update_body(x_ref, gabc_ref, w1, w2, w3, b2, b3, g, be, o_ref,
                      m_ref):
    j = pl.program_id(1)

    def make_branch(jj):
        def br():
            x = x_ref[...]
            gsum = (gabc_ref[SMAP[jj], :, 0:L]
                    + gabc_ref[DMAP[jj], :, L:2 * L])
            h = jnp.maximum(_dot(x, w1[...]) + gsum, 0.0)
            h = jnp.maximum(_dot(h, w2[...]) + b2[...], 0.0)
            y = _dot(h, w3[...]) + b3[...]
            o = x + _ln(y, g[...], be[...])
            o_ref[...] = o
            if jj < 3:
                m_ref[DMAP[jj]] = o
            else:
                m_ref[DMAP[jj]] += o
        return br

    lax.switch(j, [make_branch(jj) for jj in range(6)])


def _edge_update(el, gabc, p):
    w1e = p['W1'][0:L]
    return pl.pallas_call(
        _edge_update_body,
        grid=(NBC, 6),
        in_specs=[pl.BlockSpec((BC, L), _eblock),
                  pl.BlockSpec((3, BC, 2 * L), lambda i, j: (0, i, 0)),
                  _full((L, L)), _full((L, L)), _full((L, L)),
                  _full((1, L)), _full((1, L)), _full((1, L)), _full((1, L))],
        out_specs=[pl.BlockSpec((BC, L), _eblock),
                   pl.BlockSpec((3, BC, L), lambda i, j: (0, i, 0))],
        out_shape=[jax.ShapeDtypeStruct((E, L), _f32),
                   jax.ShapeDtypeStruct((3, CPG, L), _f32)],
    )(el, gabc, w1e, p['W2'], p['W3'], p['b2'][None], p['b3'][None],
      p['g'][None], p['be'][None])


def _node_update_body(x_ref, a0_ref, a1_ref, w1n, w1a, b1, w2, w3, b2, b3, g,
                      be, ws, wd, bd, o_ref, ppd_ref, *, proj):
    x = x_ref[...]
    a = a0_ref[0] + a1_ref[0]
    h = jnp.maximum(_dot(x, w1n[...]) + _dot(a, w1a[...]) + b1[...], 0.0)
    h = jnp.maximum(_dot(h, w2[...]) + b2[...], 0.0)
    y = _dot(h, w3[...]) + b3[...]
    o = x + _ln(y, g[...], be[...])
    o_ref[...] = o
    if proj:
        ppd_ref[:, 0:L] = _dot(o, ws[...])
        ppd_ref[:, L:2 * L] = _dot(o, wd[...]) + bd[...]


def _node_update(nl, agg, p, pe_next):
    proj = pe_next is not None
    w1n = p['W1'][0:L]
    w1a = p['W1'][L:2 * L]
    if proj:
        ws = pe_next['W1'][L:2 * L]
        wd = pe_next['W1'][2 * L:3 * L]
        bd = pe_next['b1'][None]
    else:
        ws = jnp.zeros((1, L), _f32)
        wd = jnp.zeros((1, L), _f32)
        bd = jnp.zeros((1, L), _f32)
    out_shapes = [jax.ShapeDtypeStruct((N, L), _f32),
                  jax.ShapeDtypeStruct((N, 2 * L), _f32)]
    out_specs = [pl.BlockSpec((BN, L), lambda i: (i, 0)),
                 pl.BlockSpec((BN, 2 * L), lambda i: (i, 0))]

    def body(*refs):
        if proj:
            _node_update_body(*refs, proj=True)
        else:
            _node_update_body(*refs[:16], None, proj=False)

    res = pl.pallas_call(
        body,
        grid=(N // BN,),
        in_specs=[pl.BlockSpec((BN, L), lambda i: (i, 0)),
                  pl.BlockSpec((1, BN, L), lambda i: (0, i, 0)),
                  pl.BlockSpec((1, BN, L), lambda i: (1, i, 0))] +
                 [_full((L, L)), _full((L, L)), _full((1, L)),
                  _full((L, L)), _full((L, L)), _full((1, L)), _full((1, L)),
                  _full((1, L)), _full((1, L)),
                  _full(ws.shape), _full(wd.shape), _full((1, L))],
        out_specs=out_specs if proj else out_specs[0],
        out_shape=out_shapes if proj else out_shapes[0],
    )(nl, agg, agg, w1n, w1a, p['b1'][None], p['W2'], p['W3'],
      p['b2'][None], p['b3'][None], p['g'][None], p['be'][None], ws, wd, bd)
    if proj:
        return res
    return res, None


def _node_encoder_body(wp_ref, pwp_ref, nt_ref, mean, std, w1, b1, w2, w3, b2,
                       b3, g, be, ws, wd, bd, o_ref, ppd_ref):
    vel = wp_ref[...] - pwp_ref[...]
    t = nt_ref[...]
    oh = (t == lax.broadcasted_iota(jnp.int32, (1, 9), 1)).astype(_f32)
    f = jnp.concatenate([vel, oh], axis=-1)
    f = (f - mean[...]) / std[...]
    h = jnp.maximum(_dot(f, w1[...]) + b1[...], 0.0)
    h = jnp.maximum(_dot(h, w2[...]) + b2[...], 0.0)
    y = _dot(h, w3[...]) + b3[...]
    o = _ln(y, g[...], be[...])
    o_ref[...] = o
    ppd_ref[:, 0:L] = _dot(o, ws[...])
    ppd_ref[:, L:2 * L] = _dot(o, wd[...]) + bd[...]


def _node_encoder(world_pos, prev_world_pos, node_type, params):
    p = params['node_enc']
    pe0 = params['proc_edge'][0]
    return pl.pallas_call(
        _node_encoder_body,
        grid=(N // BN,),
        in_specs=[pl.BlockSpec((BN, 3), lambda i: (i, 0)),
                  pl.BlockSpec((BN, 3), lambda i: (i, 0)),
                  pl.BlockSpec((BN, 1), lambda i: (i, 0)),
                  _full((1, 12)), _full((1, 12)),
                  _full((12, L)), _full((1, L)),
                  _full((L, L)), _full((L, L)), _full((1, L)), _full((1, L)),
                  _full((1, L)), _full((1, L)),
                  _full((L, L)), _full((L, L)), _full((1, L))],
        out_specs=[pl.BlockSpec((BN, L), lambda i: (i, 0)),
                   pl.BlockSpec((BN, 2 * L), lambda i: (i, 0))],
        out_shape=[jax.ShapeDtypeStruct((N, L), _f32),
                   jax.ShapeDtypeStruct((N, 2 * L), _f32)],
    )(world_pos, prev_world_pos, node_type[:, None],
      params['node_mean'][None], params['node_std'][None],
      p['W1'], p['b1'][None], p['W2'], p['W3'], p['b2'][None], p['b3'][None],
      p['g'][None], p['be'][None],
      pe0['W1'][L:2 * L], pe0['W1'][2 * L:3 * L], pe0['b1'][None])


def _edge_encoder_body(gt_ref, mean, std, w1, b1, w2, w3, b2, b3, g, be,
                       o_ref):
    j = pl.program_id(1)

    def make_branch(jj):
        def br():
            d = gt_ref[SMAP[jj], :, 0:5] - gt_ref[DMAP[jj], :, 0:5]
            dm = d[:, 0:2]
            dw = d[:, 2:5]
            nm = jnp.sqrt(jnp.sum(dm * dm, axis=-1, keepdims=True))
            nw = jnp.sqrt(jnp.sum(dw * dw, axis=-1, keepdims=True))
            f = jnp.concatenate([dm, nm, dw, nw], axis=-1)
            f = (f - mean[...]) / std[...]
            h = jnp.maximum(_dot(f, w1[...]) + b1[...], 0.0)
            h = jnp.maximum(_dot(h, w2[...]) + b2[...], 0.0)
            y = _dot(h, w3[...]) + b3[...]
            o_ref[...] = _ln(y, g[...], be[...])
        return br

    lax.switch(j, [make_branch(jj) for jj in range(6)])


def _edge_encoder(gt, params):
    p = params['edge_enc']
    return pl.pallas_call(
        _edge_encoder_body,
        grid=(NBC, 6),
        in_specs=[pl.BlockSpec((3, BC, L), lambda i, j: (0, i, 0)),
                  _full((1, 7)), _full((1, 7)),
                  _full((7, L)), _full((1, L)),
                  _full((L, L)), _full((L, L)), _full((1, L)), _full((1, L)),
                  _full((1, L)), _full((1, L))],
        out_specs=pl.BlockSpec((BC, L), _eblock),
        out_shape=jax.ShapeDtypeStruct((E, L), _f32),
    )(gt, params['edge_mean'][None], params['edge_std'][None],
      p['W1'], p['b1'][None], p['W2'], p['W3'], p['b2'][None], p['b3'][None],
      p['g'][None], p['be'][None])


def _decoder_body(x_ref, w1, b1, w2, b2, w3, b3, ostd, omean, o_ref):
    h = jnp.maximum(_dot(x_ref[...], w1[...]) + b1[...], 0.0)
    h = jnp.maximum(_dot(h, w2[...]) + b2[...], 0.0)
    y = _dot(h, w3[...]) + b3[...]
    o_ref[...] = y * ostd[...] + omean[...]


def _decoder(nl, params):
    p = params['decoder']
    w3p = jnp.zeros((L, L), _f32).at[:, 0:3].set(p['W3'])
    b3p = jnp.zeros((1, L), _f32).at[:, 0:3].set(p['b3'][None])
    ostd = jnp.ones((1, L), _f32).at[:, 0:3].set(params['out_std'][None])
    omean = jnp.zeros((1, L), _f32).at[:, 0:3].set(params['out_mean'][None])
    out = pl.pallas_call(
        _decoder_body,
        grid=(N // BN,),
        in_specs=[pl.BlockSpec((BN, L), lambda i: (i, 0)),
                  _full((L, L)), _full((1, L)), _full((L, L)), _full((1, L)),
                  _full((L, L)), _full((1, L)), _full((1, L)), _full((1, L))],
        out_specs=pl.BlockSpec((BN, L), lambda i: (i, 0)),
        out_shape=jax.ShapeDtypeStruct((N, L), _f32),
    )(nl, p['W1'], p['b1'][None], p['W2'], p['b2'][None], w3p, b3p, ostd,
      omean)
    return out[:, 0:3]


def _cells_idx(cells, pad_value, nchk, chunk):
    cols = []
    for k in range(3):
        cols.append(jnp.pad(cells[:, k].astype(jnp.int32), (0, CPG - C),
                            constant_values=pad_value))
    return jnp.concatenate(cols).reshape(NW, nchk, chunk)


def kernel(world_pos, prev_world_pos, target_world_pos, mesh_pos, node_type,
           cells, params):
    del target_world_pos
    gi = _cells_idx(cells, 0, NCHK_G, CHUNK_G)
    si = _cells_idx(cells, JUNK, NCHK_S, CHUNK_S)

    tbl = jnp.concatenate(
        [mesh_pos, world_pos, jnp.zeros((N, L - 5), _f32)], axis=1)
    gt = _make_sc_gather_cells(L)(tbl, gi).reshape(3, CPG, L)
    el = _edge_encoder(gt, params)
    nl, ppd = _node_encoder(world_pos, prev_world_pos, node_type, params)

    zero = jnp.zeros((ROWS_PER_TILE, L), _f32)
    gather = _make_sc_gather_cells(2 * L)
    scatter = _make_sc_scatter_add()
    for i in range(STEPS):
        gabc = gather(ppd, gi).reshape(3, CPG, 2 * L)
        el, m = _edge_update(el, gabc, params['proc_edge'][i])
        agg = scatter(m.reshape(G_ROWS, L), si, zero)
        pe_next = params['proc_edge'][i + 1] if i + 1 < STEPS else None
        nl, ppd = _node_update(nl, agg, params['proc_node'][i], pe_next)
    return _decoder(nl, params)
